# baseline (speedup 1.0000x reference)
import jax
import jax.numpy as jnp
from jax.experimental import pallas as pl

_NB = 8
_TOPK = 4
_NHEAD = 4
_HD = 16


def _rim_body(x_ref, h0_ref, wqT_ref, bq_ref, wkT_ref, bk_ref, wvT_ref, bv_ref,
              wihT_ref, bih_ref, whhT_ref, bhh_ref,
              mqT_ref, mbq_ref, mkT_ref, mbk_ref, mvT_ref, mbv_ref,
              fcT_ref, fcb_ref, out_ref):
    seq, b, _ = x_ref.shape
    nhid = h0_ref.shape[1]
    nb = _NB
    bso = nhid // nb
    att = wvT_ref.shape[1]
    kk = nb - _TOPK
    dm = _NHEAD * _HD

    bv = bv_ref[:, :]
    bk = bk_ref[:, :]

    x_all = x_ref[:, :, :].reshape(seq * b, x_ref.shape[2])
    k1_all = jnp.dot(x_all, wkT_ref[:, :]) + bk
    v1_all = jnp.dot(x_all, wvT_ref[:, :]) + bv

    hx = h0_ref[:, :]
    for t in range(seq):
        h3 = hx.reshape(b, nb, bso)

        q2 = jnp.dot(hx.reshape(b * nb, bso), wqT_ref[:, :]) + bq_ref[:, :]
        q3 = q2.reshape(b, nb, dm)
        k1 = k1_all[t * b:(t + 1) * b, :]
        v1 = v1_all[t * b:(t + 1) * b, :]
        kst = jnp.concatenate(
            [jnp.broadcast_to(bk.reshape(1, 1, dm), (b, 1, dm)), k1[:, None, :]], axis=1)
        vst = jnp.concatenate(
            [jnp.broadcast_to(bv.reshape(1, 1, att), (b, 1, att)), v1[:, None, :]], axis=1)
        lg = jnp.einsum('bqd,bkd->bqk', q3, kst) / 8.0
        at = jax.nn.softmax(lg, axis=-1)
        attv = jnp.einsum('bqk,bkd->bqd', at, vst)
        a0 = at[:, :, 0]

        av = a0[:, :, None]
        aw = a0[:, None, :]
        jj = jax.lax.broadcasted_iota(jnp.int32, (b, nb, nb), 1)
        j2 = jax.lax.broadcasted_iota(jnp.int32, (b, nb, nb), 2)
        beats = (aw > av) | ((aw == av) & (j2 < jj))
        cnt = jnp.sum(beats.astype(jnp.float32), axis=-1)
        mask = (cnt >= float(kk)).astype(jnp.float32)

        gi_list = []
        gh_list = []
        for j in range(nb):
            gij = jnp.dot(attv[:, j, :], wihT_ref[j])
            gi_list.append(gij + bih_ref[j:j + 1, :])
            ghj = jnp.dot(hx[:, j * bso:(j + 1) * bso], whhT_ref[j])
            gh_list.append(ghj + bhh_ref[j:j + 1, :])
        gi3 = jnp.stack(gi_list, axis=1)
        gh3 = jnp.stack(gh_list, axis=1)

        r = jax.nn.sigmoid(gi3[..., :bso] + gh3[..., :bso])
        z = jax.nn.sigmoid(gi3[..., bso:2 * bso] + gh3[..., bso:2 * bso])
        n = jnp.tanh(gi3[..., 2 * bso:] + r * gh3[..., 2 * bso:])
        hn3 = (1.0 - z) * n + z * h3

        hn2 = hn3.reshape(b * nb, bso)
        qm = jnp.dot(hn2, mqT_ref[:, :]) + mbq_ref[:, :]
        km = jnp.dot(hn2, mkT_ref[:, :]) + mbk_ref[:, :]
        vm = jnp.dot(hn2, mvT_ref[:, :]) + mbv_ref[:, :]
        q4 = jnp.transpose(qm.reshape(b, nb, _NHEAD, _HD),
                           (2, 0, 1, 3)).reshape(_NHEAD * b, nb, _HD)
        k4 = jnp.transpose(km.reshape(b, nb, _NHEAD, _HD),
                           (2, 0, 1, 3)).reshape(_NHEAD * b, nb, _HD)
        v4 = jnp.transpose(vm.reshape(b, nb, _NHEAD, _HD),
                           (2, 0, 1, 3)).reshape(_NHEAD * b, nb, _HD)
        lg2 = jnp.einsum('bqd,bkd->bqk', q4, k4) / 4.0
        at2 = jax.nn.softmax(lg2, axis=-1)
        o4 = jnp.einsum('bqk,bkd->bqd', at2, v4)
        om3 = jnp.transpose(o4.reshape(_NHEAD, b, nb, _HD),
                            (1, 2, 0, 3)).reshape(b, nb, dm)
        fco = jnp.dot(om3.reshape(b * nb, dm), fcT_ref[:, :]) + fcb_ref[:, :]
        fco3 = fco.reshape(b, nb, bso)

        att_h = fco3 + hn3
        hxn3 = hn3 + att_h
        m3 = mask[:, :, None]
        hx3 = m3 * hxn3 + (1.0 - m3) * h3
        hx = hx3.reshape(b, nhid)
        out_ref[t] = hx


def kernel(input, hidden, seq_len, inp_qW, inp_qb, inp_kW, inp_kb, inp_vW, inp_vb,
           mha_qW, mha_qb, mha_kW, mha_kb, mha_vW, mha_vb, mha_fcW, mha_fcb,
           gru_Wih, gru_Whh, gru_bih, gru_bhh):
    seq, b, ninp = input.shape
    nhid = hidden.shape[1]
    nb = _NB
    bso = nhid // nb
    att = inp_vW.shape[0]

    def blk(W, j, C):
        return jnp.concatenate(
            [W[g * nhid + j * bso:g * nhid + (j + 1) * bso, j * C:(j + 1) * C]
             for g in range(3)], axis=0)

    wihT = jnp.transpose(
        jnp.stack([blk(gru_Wih, j, att) for j in range(nb)]), (0, 2, 1))
    whhT = jnp.transpose(
        jnp.stack([blk(gru_Whh, j, bso) for j in range(nb)]), (0, 2, 1))
    bihc = gru_bih.reshape(3, nb, bso).transpose(1, 0, 2).reshape(nb, 3 * bso)
    bhhc = gru_bhh.reshape(3, nb, bso).transpose(1, 0, 2).reshape(nb, 3 * bso)

    args = (
        input, hidden,
        inp_qW.T, inp_qb.reshape(1, -1),
        inp_kW.T, inp_kb.reshape(1, -1),
        inp_vW.T, inp_vb.reshape(1, -1),
        wihT, bihc, whhT, bhhc,
        mha_qW.T, mha_qb.reshape(1, -1),
        mha_kW.T, mha_kb.reshape(1, -1),
        mha_vW.T, mha_vb.reshape(1, -1),
        mha_fcW.T, mha_fcb.reshape(1, -1),
    )
    out = pl.pallas_call(
        _rim_body,
        out_shape=jax.ShapeDtypeStruct((seq, b, nhid), jnp.float32),
    )(*args)
    return (out, out[-1])


# speedup vs baseline: 1.0773x; 1.0773x over previous
import jax
import jax.numpy as jnp
from jax.experimental import pallas as pl

_NB = 8
_TOPK = 4
_NHEAD = 4
_HD = 16


def _rim_body(x_ref, h0_ref, wqT_ref, bq_ref, wkT_ref, bk_ref, wvT_ref, bv_ref,
              wihT_ref, bih_ref, whhT_ref, bhh_ref,
              mqT_ref, mbq_ref, mkT_ref, mbk_ref, mvT_ref, mbv_ref,
              fcT_ref, fcb_ref, out_ref, hf_ref):
    seq, b, _ = x_ref.shape
    nhid = h0_ref.shape[1]
    nb = _NB
    bso = nhid // nb
    att = wvT_ref.shape[1]
    kk = nb - _TOPK
    dm = _NHEAD * _HD

    bv = bv_ref[:, :]
    bk = bk_ref[:, :]

    def step(t, hx):
        x = x_ref[pl.ds(t, 1)].reshape(b, x_ref.shape[2])
        h3 = hx.reshape(b, nb, bso)

        q2 = jnp.dot(hx.reshape(b * nb, bso), wqT_ref[:, :]) + bq_ref[:, :]
        q3 = q2.reshape(b, nb, dm)
        k1 = jnp.dot(x, wkT_ref[:, :]) + bk
        v1 = jnp.dot(x, wvT_ref[:, :]) + bv
        kst = jnp.concatenate(
            [jnp.broadcast_to(bk.reshape(1, 1, dm), (b, 1, dm)), k1[:, None, :]], axis=1)
        vst = jnp.concatenate(
            [jnp.broadcast_to(bv.reshape(1, 1, att), (b, 1, att)), v1[:, None, :]], axis=1)
        lg = jnp.einsum('bqd,bkd->bqk', q3, kst) / 8.0
        at = jax.nn.softmax(lg, axis=-1)
        attv = jnp.einsum('bqk,bkd->bqd', at, vst)
        a0 = at[:, :, 0]

        av = a0[:, :, None]
        aw = a0[:, None, :]
        jj = jax.lax.broadcasted_iota(jnp.int32, (b, nb, nb), 1)
        j2 = jax.lax.broadcasted_iota(jnp.int32, (b, nb, nb), 2)
        beats = (aw > av) | ((aw == av) & (j2 < jj))
        cnt = jnp.sum(beats.astype(jnp.float32), axis=-1)
        mask = (cnt >= float(kk)).astype(jnp.float32)

        gi_list = []
        gh_list = []
        dnum = (((1,), (1,)), ((), ()))
        for j in range(nb):
            gij = jax.lax.dot_general(attv[:, j, :], wihT_ref[j], dnum)
            gi_list.append(gij + bih_ref[j:j + 1, :])
            ghj = jax.lax.dot_general(hx[:, j * bso:(j + 1) * bso], whhT_ref[j], dnum)
            gh_list.append(ghj + bhh_ref[j:j + 1, :])
        gi3 = jnp.stack(gi_list, axis=1)
        gh3 = jnp.stack(gh_list, axis=1)

        r = jax.nn.sigmoid(gi3[..., :bso] + gh3[..., :bso])
        z = jax.nn.sigmoid(gi3[..., bso:2 * bso] + gh3[..., bso:2 * bso])
        n = jnp.tanh(gi3[..., 2 * bso:] + r * gh3[..., 2 * bso:])
        hn3 = (1.0 - z) * n + z * h3

        hn2 = hn3.reshape(b * nb, bso)
        qm = jnp.dot(hn2, mqT_ref[:, :]) + mbq_ref[:, :]
        km = jnp.dot(hn2, mkT_ref[:, :]) + mbk_ref[:, :]
        vm = jnp.dot(hn2, mvT_ref[:, :]) + mbv_ref[:, :]
        q4 = jnp.transpose(qm.reshape(b, nb, _NHEAD, _HD),
                           (2, 0, 1, 3)).reshape(_NHEAD * b, nb, _HD)
        k4 = jnp.transpose(km.reshape(b, nb, _NHEAD, _HD),
                           (2, 0, 1, 3)).reshape(_NHEAD * b, nb, _HD)
        v4 = jnp.transpose(vm.reshape(b, nb, _NHEAD, _HD),
                           (2, 0, 1, 3)).reshape(_NHEAD * b, nb, _HD)
        lg2 = jnp.einsum('bqd,bkd->bqk', q4, k4) / 4.0
        at2 = jax.nn.softmax(lg2, axis=-1)
        o4 = jnp.einsum('bqk,bkd->bqd', at2, v4)
        om3 = jnp.transpose(o4.reshape(_NHEAD, b, nb, _HD),
                            (1, 2, 0, 3)).reshape(b, nb, dm)
        fco = jnp.dot(om3.reshape(b * nb, dm), fcT_ref[:, :]) + fcb_ref[:, :]
        fco3 = fco.reshape(b, nb, bso)

        att_h = fco3 + hn3
        hxn3 = hn3 + att_h
        m3 = mask[:, :, None]
        hx3 = m3 * hxn3 + (1.0 - m3) * h3
        hxn = hx3.reshape(b, nhid)
        out_ref[pl.ds(t, 1)] = hxn.reshape(1, b, nhid)
        return hxn

    hf_ref[:, :] = jax.lax.fori_loop(0, seq, step, h0_ref[:, :])


def kernel(input, hidden, seq_len, inp_qW, inp_qb, inp_kW, inp_kb, inp_vW, inp_vb,
           mha_qW, mha_qb, mha_kW, mha_kb, mha_vW, mha_vb, mha_fcW, mha_fcb,
           gru_Wih, gru_Whh, gru_bih, gru_bhh):
    seq, b, ninp = input.shape
    nhid = hidden.shape[1]
    nb = _NB
    bso = nhid // nb
    att = inp_vW.shape[0]

    def blk(W, j, C):
        return jnp.concatenate(
            [W[g * nhid + j * bso:g * nhid + (j + 1) * bso, j * C:(j + 1) * C]
             for g in range(3)], axis=0)

    wihT = jnp.stack([blk(gru_Wih, j, att) for j in range(nb)])
    whhT = jnp.stack([blk(gru_Whh, j, bso) for j in range(nb)])
    bihc = gru_bih.reshape(3, nb, bso).transpose(1, 0, 2).reshape(nb, 3 * bso)
    bhhc = gru_bhh.reshape(3, nb, bso).transpose(1, 0, 2).reshape(nb, 3 * bso)

    args = (
        input, hidden,
        inp_qW.T, inp_qb.reshape(1, -1),
        inp_kW.T, inp_kb.reshape(1, -1),
        inp_vW.T, inp_vb.reshape(1, -1),
        wihT, bihc, whhT, bhhc,
        mha_qW.T, mha_qb.reshape(1, -1),
        mha_kW.T, mha_kb.reshape(1, -1),
        mha_vW.T, mha_vb.reshape(1, -1),
        mha_fcW.T, mha_fcb.reshape(1, -1),
    )
    out, hxf = pl.pallas_call(
        _rim_body,
        out_shape=(jax.ShapeDtypeStruct((seq, b, nhid), jnp.float32),
                   jax.ShapeDtypeStruct((b, nhid), jnp.float32)),
    )(*args)
    return (out, hxf)


# speedup vs baseline: 1.1126x; 1.0327x over previous
import jax
import jax.numpy as jnp
from jax.experimental import pallas as pl

_NB = 8
_TOPK = 4
_NHEAD = 4
_HD = 16


def _rim_body(x_ref, h0_ref, wqT_ref, bq_ref, wkT_ref, bk_ref, wvT_ref, bv_ref,
              wihT_ref, bih_ref, whhT_ref, bhh_ref,
              mqT_ref, mbq_ref, mkT_ref, mbk_ref, mvT_ref, mbv_ref,
              fcT_ref, fcb_ref, out_ref, hf_ref):
    seq, b, _ = x_ref.shape
    nhid = h0_ref.shape[1]
    nb = _NB
    bso = nhid // nb
    att = wvT_ref.shape[0]
    kk = nb - _TOPK
    dm = _NHEAD * _HD

    bv = bv_ref[:, :]
    bk = bk_ref[:, :]
    dn1 = (((1,), (1,)), ((), ()))

    def step(t, hx):
        x = x_ref[pl.ds(t, 1)].reshape(b, x_ref.shape[2])
        h3 = hx.reshape(b, nb, bso)

        q2 = jax.lax.dot_general(hx.reshape(b * nb, bso), wqT_ref[:, :], dn1) + bq_ref[:, :]
        q3 = q2.reshape(b, nb, dm)
        k1 = jax.lax.dot_general(x, wkT_ref[:, :], dn1) + bk
        v1 = jax.lax.dot_general(x, wvT_ref[:, :], dn1) + bv
        kst = jnp.concatenate(
            [jnp.broadcast_to(bk.reshape(1, 1, dm), (b, 1, dm)), k1[:, None, :]], axis=1)
        vst = jnp.concatenate(
            [jnp.broadcast_to(bv.reshape(1, 1, att), (b, 1, att)), v1[:, None, :]], axis=1)
        lg = jnp.einsum('bqd,bkd->bqk', q3, kst) / 8.0
        at = jax.nn.softmax(lg, axis=-1)
        attv = jnp.einsum('bqk,bkd->bqd', at, vst)
        a0 = at[:, :, 0]

        av = a0[:, :, None]
        aw = a0[:, None, :]
        jj = jax.lax.broadcasted_iota(jnp.int32, (b, nb, nb), 1)
        j2 = jax.lax.broadcasted_iota(jnp.int32, (b, nb, nb), 2)
        beats = (aw > av) | ((aw == av) & (j2 < jj))
        cnt = jnp.sum(beats.astype(jnp.float32), axis=-1)
        mask = (cnt >= float(kk)).astype(jnp.float32)

        gi_list = []
        gh_list = []
        for j in range(nb):
            gij = jax.lax.dot_general(attv[:, j, :], wihT_ref[j], dn1)
            gi_list.append(gij + bih_ref[j:j + 1, :])
            ghj = jax.lax.dot_general(hx[:, j * bso:(j + 1) * bso], whhT_ref[j], dn1)
            gh_list.append(ghj + bhh_ref[j:j + 1, :])
        gi3 = jnp.stack(gi_list, axis=1)
        gh3 = jnp.stack(gh_list, axis=1)

        r = jax.nn.sigmoid(gi3[..., :bso] + gh3[..., :bso])
        z = jax.nn.sigmoid(gi3[..., bso:2 * bso] + gh3[..., bso:2 * bso])
        n = jnp.tanh(gi3[..., 2 * bso:] + r * gh3[..., 2 * bso:])
        hn3 = (1.0 - z) * n + z * h3

        hn2 = hn3.reshape(b * nb, bso)
        qm = jax.lax.dot_general(hn2, mqT_ref[:, :], dn1) + mbq_ref[:, :]
        km = jax.lax.dot_general(hn2, mkT_ref[:, :], dn1) + mbk_ref[:, :]
        vm = jax.lax.dot_general(hn2, mvT_ref[:, :], dn1) + mbv_ref[:, :]
        q4 = jnp.transpose(qm.reshape(b, nb, _NHEAD, _HD),
                           (2, 0, 1, 3)).reshape(_NHEAD * b, nb, _HD)
        k4 = jnp.transpose(km.reshape(b, nb, _NHEAD, _HD),
                           (2, 0, 1, 3)).reshape(_NHEAD * b, nb, _HD)
        v4 = jnp.transpose(vm.reshape(b, nb, _NHEAD, _HD),
                           (2, 0, 1, 3)).reshape(_NHEAD * b, nb, _HD)
        lg2 = jnp.einsum('bqd,bkd->bqk', q4, k4) / 4.0
        at2 = jax.nn.softmax(lg2, axis=-1)
        o4 = jnp.einsum('bqk,bkd->bqd', at2, v4)
        om3 = jnp.transpose(o4.reshape(_NHEAD, b, nb, _HD),
                            (1, 2, 0, 3)).reshape(b, nb, dm)
        fco = jax.lax.dot_general(om3.reshape(b * nb, dm), fcT_ref[:, :], dn1) + fcb_ref[:, :]
        fco3 = fco.reshape(b, nb, bso)

        att_h = fco3 + hn3
        hxn3 = hn3 + att_h
        m3 = mask[:, :, None]
        hx3 = m3 * hxn3 + (1.0 - m3) * h3
        hxn = hx3.reshape(b, nhid)
        out_ref[pl.ds(t, 1)] = hxn.reshape(1, b, nhid)
        return hxn

    hf_ref[:, :] = jax.lax.fori_loop(0, seq, step, h0_ref[:, :])


def kernel(input, hidden, seq_len, inp_qW, inp_qb, inp_kW, inp_kb, inp_vW, inp_vb,
           mha_qW, mha_qb, mha_kW, mha_kb, mha_vW, mha_vb, mha_fcW, mha_fcb,
           gru_Wih, gru_Whh, gru_bih, gru_bhh):
    seq, b, ninp = input.shape
    nhid = hidden.shape[1]
    nb = _NB
    bso = nhid // nb
    att = inp_vW.shape[0]

    def blk(W, j, C):
        return jnp.concatenate(
            [W[g * nhid + j * bso:g * nhid + (j + 1) * bso, j * C:(j + 1) * C]
             for g in range(3)], axis=0)

    wihT = jnp.stack([blk(gru_Wih, j, att) for j in range(nb)])
    whhT = jnp.stack([blk(gru_Whh, j, bso) for j in range(nb)])
    bihc = gru_bih.reshape(3, nb, bso).transpose(1, 0, 2).reshape(nb, 3 * bso)
    bhhc = gru_bhh.reshape(3, nb, bso).transpose(1, 0, 2).reshape(nb, 3 * bso)

    args = (
        input, hidden,
        inp_qW, inp_qb.reshape(1, -1),
        inp_kW, inp_kb.reshape(1, -1),
        inp_vW, inp_vb.reshape(1, -1),
        wihT, bihc, whhT, bhhc,
        mha_qW, mha_qb.reshape(1, -1),
        mha_kW, mha_kb.reshape(1, -1),
        mha_vW, mha_vb.reshape(1, -1),
        mha_fcW, mha_fcb.reshape(1, -1),
    )
    out, hxf = pl.pallas_call(
        _rim_body,
        out_shape=(jax.ShapeDtypeStruct((seq, b, nhid), jnp.float32),
                   jax.ShapeDtypeStruct((b, nhid), jnp.float32)),
    )(*args)
    return (out, hxf)


# speedup vs baseline: 1.1368x; 1.0218x over previous
import jax
import jax.numpy as jnp
from jax.experimental import pallas as pl

_NB = 8
_TOPK = 4
_NHEAD = 4
_HD = 16


def _rim_body(x_ref, h0_ref, wqT_ref, bq_ref, wkT_ref, bk_ref, wvT_ref, bv_ref,
              wihT_ref, bih_ref, whhT_ref, bhh_ref,
              mqT_ref, mbq_ref, mkT_ref, mbk_ref, mvT_ref, mbv_ref,
              fcT_ref, fcb_ref, out_ref, hf_ref):
    seq, b, _ = x_ref.shape
    nhid = h0_ref.shape[1]
    nb = _NB
    bso = nhid // nb
    att = wvT_ref.shape[0]
    kk = nb - _TOPK
    dm = _NHEAD * _HD

    bv = bv_ref[:, :]
    bk = bk_ref[:, :]
    dn1 = (((1,), (1,)), ((), ()))

    def step(t, hx):
        x = x_ref[pl.ds(t, 1)].reshape(b, x_ref.shape[2])
        h3 = hx.reshape(b, nb, bso)

        q2 = jax.lax.dot_general(hx.reshape(b * nb, bso), wqT_ref[:, :], dn1) + bq_ref[:, :]
        q3 = q2.reshape(b, nb, dm)
        k1 = jax.lax.dot_general(x, wkT_ref[:, :], dn1) + bk
        v1 = jax.lax.dot_general(x, wvT_ref[:, :], dn1) + bv
        kst = jnp.concatenate(
            [jnp.broadcast_to(bk.reshape(1, 1, dm), (b, 1, dm)), k1[:, None, :]], axis=1)
        vst = jnp.concatenate(
            [jnp.broadcast_to(bv.reshape(1, 1, att), (b, 1, att)), v1[:, None, :]], axis=1)
        lg = jnp.einsum('bqd,bkd->bqk', q3, kst) / 8.0
        at = jax.nn.softmax(lg, axis=-1)
        attv = jnp.einsum('bqk,bkd->bqd', at, vst)
        a0 = at[:, :, 0]

        av = a0[:, :, None]
        aw = a0[:, None, :]
        jj = jax.lax.broadcasted_iota(jnp.int32, (b, nb, nb), 1)
        j2 = jax.lax.broadcasted_iota(jnp.int32, (b, nb, nb), 2)
        beats = (aw > av) | ((aw == av) & (j2 < jj))
        cnt = jnp.sum(beats.astype(jnp.float32), axis=-1)
        mask = (cnt >= float(kk)).astype(jnp.float32)

        gi_list = []
        gh_list = []
        for j in range(nb):
            gij = jax.lax.dot_general(attv[:, j, :], wihT_ref[j], dn1)
            gi_list.append(gij + bih_ref[j:j + 1, :])
            ghj = jax.lax.dot_general(hx[:, j * bso:(j + 1) * bso], whhT_ref[j], dn1)
            gh_list.append(ghj + bhh_ref[j:j + 1, :])
        gi3 = jnp.stack(gi_list, axis=1)
        gh3 = jnp.stack(gh_list, axis=1)

        r = jax.nn.sigmoid(gi3[..., :bso] + gh3[..., :bso])
        z = jax.nn.sigmoid(gi3[..., bso:2 * bso] + gh3[..., bso:2 * bso])
        n = jnp.tanh(gi3[..., 2 * bso:] + r * gh3[..., 2 * bso:])
        hn3 = (1.0 - z) * n + z * h3

        hn2 = hn3.reshape(b * nb, bso)
        o_heads = []
        for h in range(_NHEAD):
            rs = slice(h * _HD, (h + 1) * _HD)
            qh = (jax.lax.dot_general(hn2, mqT_ref[rs, :], dn1)
                  + mbq_ref[:, rs]).reshape(b, nb, _HD)
            kh = (jax.lax.dot_general(hn2, mkT_ref[rs, :], dn1)
                  + mbk_ref[:, rs]).reshape(b, nb, _HD)
            vh = (jax.lax.dot_general(hn2, mvT_ref[rs, :], dn1)
                  + mbv_ref[:, rs]).reshape(b, nb, _HD)
            lg2 = jnp.einsum('bqd,bkd->bqk', qh, kh) / 4.0
            at2 = jax.nn.softmax(lg2, axis=-1)
            o_heads.append(jnp.einsum('bqk,bkd->bqd', at2, vh))
        om3 = jnp.concatenate(o_heads, axis=-1)
        fco = jax.lax.dot_general(om3.reshape(b * nb, dm), fcT_ref[:, :], dn1) + fcb_ref[:, :]
        fco3 = fco.reshape(b, nb, bso)

        att_h = fco3 + hn3
        hxn3 = hn3 + att_h
        m3 = mask[:, :, None]
        hx3 = m3 * hxn3 + (1.0 - m3) * h3
        hxn = hx3.reshape(b, nhid)
        out_ref[pl.ds(t, 1)] = hxn.reshape(1, b, nhid)
        return hxn

    hf_ref[:, :] = jax.lax.fori_loop(0, seq, step, h0_ref[:, :])


def kernel(input, hidden, seq_len, inp_qW, inp_qb, inp_kW, inp_kb, inp_vW, inp_vb,
           mha_qW, mha_qb, mha_kW, mha_kb, mha_vW, mha_vb, mha_fcW, mha_fcb,
           gru_Wih, gru_Whh, gru_bih, gru_bhh):
    seq, b, ninp = input.shape
    nhid = hidden.shape[1]
    nb = _NB
    bso = nhid // nb
    att = inp_vW.shape[0]

    def blk(W, j, C):
        return jnp.concatenate(
            [W[g * nhid + j * bso:g * nhid + (j + 1) * bso, j * C:(j + 1) * C]
             for g in range(3)], axis=0)

    wihT = jnp.stack([blk(gru_Wih, j, att) for j in range(nb)])
    whhT = jnp.stack([blk(gru_Whh, j, bso) for j in range(nb)])
    bihc = gru_bih.reshape(3, nb, bso).transpose(1, 0, 2).reshape(nb, 3 * bso)
    bhhc = gru_bhh.reshape(3, nb, bso).transpose(1, 0, 2).reshape(nb, 3 * bso)

    args = (
        input, hidden,
        inp_qW, inp_qb.reshape(1, -1),
        inp_kW, inp_kb.reshape(1, -1),
        inp_vW, inp_vb.reshape(1, -1),
        wihT, bihc, whhT, bhhc,
        mha_qW, mha_qb.reshape(1, -1),
        mha_kW, mha_kb.reshape(1, -1),
        mha_vW, mha_vb.reshape(1, -1),
        mha_fcW, mha_fcb.reshape(1, -1),
    )
    out, hxf = pl.pallas_call(
        _rim_body,
        out_shape=(jax.ShapeDtypeStruct((seq, b, nhid), jnp.float32),
                   jax.ShapeDtypeStruct((b, nhid), jnp.float32)),
    )(*args)
    return (out, hxf)


# speedup vs baseline: 1.2178x; 1.0712x over previous
import jax
import jax.numpy as jnp
from jax.experimental import pallas as pl
from jax.experimental.pallas import tpu as pltpu

_NB = 8
_TOPK = 4
_NHEAD = 4
_HD = 16


def _rim_body(x_ref, h0_ref, wqT_ref, bq_ref, wkT_ref, bk_ref, wvT_ref, bv_ref,
              wih_hbm, bih_ref, whh_hbm, bhh_ref,
              mqT_ref, mbq_ref, mkT_ref, mbk_ref, mvT_ref, mbv_ref,
              fcT_ref, fcb_ref, out_ref, hf_ref, wih_s, whh_s, sem):
    seq, b, _ = x_ref.shape
    nhid = h0_ref.shape[1]
    nb = _NB
    bso = nhid // nb
    att = wvT_ref.shape[0]

    cps = []
    for j in range(_NB):
        for g in range(3):
            cp = pltpu.make_async_copy(
                wih_hbm.at[pl.ds(g * nhid + j * bso, bso), pl.ds(j * att, att)],
                wih_s.at[j, pl.ds(g * bso, bso), :], sem)
            cp.start()
            cps.append(cp)
            cp = pltpu.make_async_copy(
                whh_hbm.at[pl.ds(g * nhid + j * bso, bso), pl.ds(j * bso, bso)],
                whh_s.at[j, pl.ds(g * bso, bso), :], sem)
            cp.start()
            cps.append(cp)
    for cp in cps:
        cp.wait()
    kk = nb - _TOPK
    dm = _NHEAD * _HD

    bv = bv_ref[:, :]
    bk = bk_ref[:, :]
    dn1 = (((1,), (1,)), ((), ()))

    def step(t, hx):
        x = x_ref[pl.ds(t, 1)].reshape(b, x_ref.shape[2])
        h3 = hx.reshape(b, nb, bso)

        q2 = jax.lax.dot_general(hx.reshape(b * nb, bso), wqT_ref[:, :], dn1) + bq_ref[:, :]
        q3 = q2.reshape(b, nb, dm)
        k1 = jax.lax.dot_general(x, wkT_ref[:, :], dn1) + bk
        v1 = jax.lax.dot_general(x, wvT_ref[:, :], dn1) + bv
        kst = jnp.concatenate(
            [jnp.broadcast_to(bk.reshape(1, 1, dm), (b, 1, dm)), k1[:, None, :]], axis=1)
        vst = jnp.concatenate(
            [jnp.broadcast_to(bv.reshape(1, 1, att), (b, 1, att)), v1[:, None, :]], axis=1)
        lg = jnp.einsum('bqd,bkd->bqk', q3, kst) / 8.0
        at = jax.nn.softmax(lg, axis=-1)
        attv = jnp.einsum('bqk,bkd->bqd', at, vst)
        a0 = at[:, :, 0]

        av = a0[:, :, None]
        aw = a0[:, None, :]
        jj = jax.lax.broadcasted_iota(jnp.int32, (b, nb, nb), 1)
        j2 = jax.lax.broadcasted_iota(jnp.int32, (b, nb, nb), 2)
        beats = (aw > av) | ((aw == av) & (j2 < jj))
        cnt = jnp.sum(beats.astype(jnp.float32), axis=-1)
        mask = (cnt >= float(kk)).astype(jnp.float32)

        gi_list = []
        gh_list = []
        for j in range(nb):
            gij = jax.lax.dot_general(attv[:, j, :], wih_s[j], dn1)
            gi_list.append(gij + bih_ref[j:j + 1, :])
            ghj = jax.lax.dot_general(hx[:, j * bso:(j + 1) * bso], whh_s[j], dn1)
            gh_list.append(ghj + bhh_ref[j:j + 1, :])
        gi3 = jnp.stack(gi_list, axis=1)
        gh3 = jnp.stack(gh_list, axis=1)

        r = jax.nn.sigmoid(gi3[..., :bso] + gh3[..., :bso])
        z = jax.nn.sigmoid(gi3[..., bso:2 * bso] + gh3[..., bso:2 * bso])
        n = jnp.tanh(gi3[..., 2 * bso:] + r * gh3[..., 2 * bso:])
        hn3 = (1.0 - z) * n + z * h3

        hn2 = hn3.reshape(b * nb, bso)
        o_heads = []
        for h in range(_NHEAD):
            rs = slice(h * _HD, (h + 1) * _HD)
            qh = (jax.lax.dot_general(hn2, mqT_ref[rs, :], dn1)
                  + mbq_ref[:, rs]).reshape(b, nb, _HD)
            kh = (jax.lax.dot_general(hn2, mkT_ref[rs, :], dn1)
                  + mbk_ref[:, rs]).reshape(b, nb, _HD)
            vh = (jax.lax.dot_general(hn2, mvT_ref[rs, :], dn1)
                  + mbv_ref[:, rs]).reshape(b, nb, _HD)
            lg2 = jnp.einsum('bqd,bkd->bqk', qh, kh) / 4.0
            at2 = jax.nn.softmax(lg2, axis=-1)
            o_heads.append(jnp.einsum('bqk,bkd->bqd', at2, vh))
        om3 = jnp.concatenate(o_heads, axis=-1)
        fco = jax.lax.dot_general(om3.reshape(b * nb, dm), fcT_ref[:, :], dn1) + fcb_ref[:, :]
        fco3 = fco.reshape(b, nb, bso)

        att_h = fco3 + hn3
        hxn3 = hn3 + att_h
        m3 = mask[:, :, None]
        hx3 = m3 * hxn3 + (1.0 - m3) * h3
        hxn = hx3.reshape(b, nhid)
        out_ref[pl.ds(t, 1)] = hxn.reshape(1, b, nhid)
        return hxn

    hf_ref[:, :] = jax.lax.fori_loop(0, seq, step, h0_ref[:, :])


def kernel(input, hidden, seq_len, inp_qW, inp_qb, inp_kW, inp_kb, inp_vW, inp_vb,
           mha_qW, mha_qb, mha_kW, mha_kb, mha_vW, mha_vb, mha_fcW, mha_fcb,
           gru_Wih, gru_Whh, gru_bih, gru_bhh):
    seq, b, ninp = input.shape
    nhid = hidden.shape[1]
    nb = _NB
    bso = nhid // nb
    att = inp_vW.shape[0]

    bihc = gru_bih.reshape(3, nb, bso).transpose(1, 0, 2).reshape(nb, 3 * bso)
    bhhc = gru_bhh.reshape(3, nb, bso).transpose(1, 0, 2).reshape(nb, 3 * bso)

    args = (
        input, hidden,
        inp_qW, inp_qb.reshape(1, -1),
        inp_kW, inp_kb.reshape(1, -1),
        inp_vW, inp_vb.reshape(1, -1),
        gru_Wih, bihc, gru_Whh, bhhc,
        mha_qW, mha_qb.reshape(1, -1),
        mha_kW, mha_kb.reshape(1, -1),
        mha_vW, mha_vb.reshape(1, -1),
        mha_fcW, mha_fcb.reshape(1, -1),
    )
    vspec = pl.BlockSpec(memory_space=pltpu.MemorySpace.VMEM)
    hspec = pl.BlockSpec(memory_space=pltpu.MemorySpace.HBM)
    in_specs = [vspec] * 20
    in_specs[8] = hspec
    in_specs[10] = hspec
    out, hxf = pl.pallas_call(
        _rim_body,
        out_shape=(jax.ShapeDtypeStruct((seq, b, nhid), jnp.float32),
                   jax.ShapeDtypeStruct((b, nhid), jnp.float32)),
        in_specs=in_specs,
        scratch_shapes=[
            pltpu.VMEM((nb, 3 * bso, att), jnp.float32),
            pltpu.VMEM((nb, 3 * bso, bso), jnp.float32),
            pltpu.SemaphoreType.DMA,
        ],
    )(*args)
    return (out, hxf)
